# Initial kernel scaffold; baseline (speedup 1.0000x reference)
import jax
import jax.numpy as jnp
from jax.experimental import pallas as pl


def kernel(data, segment_ids, exclude_mask):
    raise NotImplementedError("write your pallas kernel here")


# speedup vs baseline: 1.9621x; 1.9621x over previous
import jax
import jax.numpy as jnp
from jax import lax
from jax.experimental import pallas as pl
from jax.experimental.pallas import tpu as pltpu
from jax.experimental.pallas import tpu_sc as plsc

N = 320000
D = 128
S = 10000

L = 16
NV = D // L
NC = 2
NS = 16
NW = NC * NS
C = N // NW
RB = 200
NBLK = N // RB
BPW = C // RB

NEG_INF = float("-inf")


def _sc_body(data_hbm, ids_hbm, excl_hbm, out_hbm,
             datbuf, idsbuf, exclbuf, stagebuf, zerobuf, prevbuf, accbuf, st):
    wid = lax.axis_index("s") * NC + lax.axis_index("c")
    start = wid * C
    end = start + C
    g0 = wid * BPW

    zv = jnp.zeros((L,), jnp.float32)
    ninf = jnp.full((L,), NEG_INF, jnp.float32)
    for k in range(NV):
        zerobuf[k * L:(k + 1) * L] = zv
        accbuf[k * L:(k + 1) * L] = ninf

    st[0] = jnp.int32(0)
    st[1] = jnp.int32(-1)
    st[2] = jnp.int32(0)
    st[3] = jnp.int32(S)

    prevbuf[0:L] = jnp.full((L,), -1, jnp.int32)

    @pl.when(wid > 0)
    def _():
        pltpu.sync_copy(ids_hbm.at[pl.ds(start - 8, 8)], prevbuf.at[pl.ds(0, 8)])

    prev = jnp.where(wid > 0, prevbuf[0:L][7], jnp.int32(-1))

    def flush_gap(cur, nxt, accs):
        @pl.when(cur >= 0)
        def _():
            for k in range(NV):
                stagebuf[k * L:(k + 1) * L] = accs[k]
            pltpu.sync_copy(stagebuf, out_hbm.at[cur])

        lo = jnp.where(cur >= 0, cur + 1,
                       jnp.where(wid == 0, jnp.int32(0), nxt))

        def gap_body(s, c):
            pltpu.sync_copy(zerobuf, out_hbm.at[s])
            return c

        lax.fori_loop(lo, nxt, gap_body, jnp.int32(0))

    def outer_body(g, carry):
        @pl.when(st[0] == 0)
        def _():
            base = g * RB
            pltpu.sync_copy(data_hbm.at[pl.ds(base, RB), :], datbuf)
            pltpu.sync_copy(ids_hbm.at[pl.ds(base, RB)],
                            idsbuf.at[pl.ds(0, RB)])
            pltpu.sync_copy(excl_hbm.at[pl.ds(base, RB)],
                            exclbuf.at[pl.ds(0, RB)])

            cur0 = st[1]
            own0 = st[2] != 0
            nf0 = st[3]
            accs0 = tuple(accbuf[k * L:(k + 1) * L] for k in range(NV))

            def body(i, carry):
                cur, own, done, nf, accs = carry
                r = base + i
                sid = idsbuf[pl.ds(i, L)][0]
                e = exclbuf[pl.ds(i, L)][0]
                is_new = sid != cur
                not_own = jnp.logical_not(own)
                past = r >= end
                not_done = jnp.logical_not(done)
                stop_now = past & (not_own | is_new) & not_done
                act = not_done & jnp.logical_not(stop_now)
                skip = not_own & (sid == prev) & act
                go = act & jnp.logical_not(skip)
                trans = go & is_new

                @pl.when(trans)
                def _():
                    flush_gap(cur, sid, accs)

                m = (jnp.int32(1) - e).astype(jnp.float32)
                new_accs = []
                for k in range(NV):
                    a = jnp.where(trans, ninf, accs[k])
                    contrib = datbuf[i, k * L:(k + 1) * L] * m
                    new_accs.append(jnp.where(go, jnp.maximum(a, contrib), a))
                cur2 = jnp.where(trans, sid, cur)
                own2 = own | trans
                done2 = done | stop_now
                nf2 = jnp.where(stop_now, sid, nf)
                return (cur2, own2, done2, nf2, tuple(new_accs))

            carry0 = (cur0, own0, jnp.bool_(False), nf0, accs0)
            cur, own, done, nf, accs = lax.fori_loop(0, RB, body, carry0)

            st[1] = cur
            st[2] = own.astype(jnp.int32)
            st[3] = nf
            st[0] = done.astype(jnp.int32)
            for k in range(NV):
                accbuf[k * L:(k + 1) * L] = accs[k]

        return carry

    lax.fori_loop(g0, NBLK, outer_body, jnp.int32(0))

    cur_f = st[1]
    nf_f = st[3]
    accs_f = tuple(accbuf[k * L:(k + 1) * L] for k in range(NV))
    flush_gap(cur_f, nf_f, accs_f)


@jax.jit
def kernel(data, segment_ids, exclude_mask):
    ids = segment_ids.astype(jnp.int32)
    excl = exclude_mask.astype(jnp.int32)
    mesh = plsc.VectorSubcoreMesh(core_axis_name="c", subcore_axis_name="s")
    run = pl.kernel(
        _sc_body,
        mesh=mesh,
        out_type=jax.ShapeDtypeStruct((S, D), jnp.float32),
        scratch_types=[
            pltpu.VMEM((RB, D), jnp.float32),
            pltpu.VMEM((RB + L,), jnp.int32),
            pltpu.VMEM((RB + L,), jnp.int32),
            pltpu.VMEM((D,), jnp.float32),
            pltpu.VMEM((D,), jnp.float32),
            pltpu.VMEM((L,), jnp.int32),
            pltpu.VMEM((D,), jnp.float32),
            pltpu.SMEM((8,), jnp.int32),
        ],
    )
    return run(data, ids, excl)
